# baseline (speedup 1.0000x reference)
import jax
import jax.numpy as jnp
from jax import lax
from jax.experimental import pallas as pl
from jax.experimental.pallas import tpu as pltpu

H = 128
W = 128
N = 2048

BAND_ROWS = 8
NBANDS = H // BAND_ROWS
P = BAND_ROWS * W
K = 128
NCK = N // K
ALPHA_MIN = 1.0 / 255.0
CULL_SLACK = 0.01


def _raster_kernel(cnt_ref, geom, colp, ucovxx, ucovxy, ucovyy, bg,
                   color_out, radii_out, pcp_out):
    b = pl.program_id(0)

    @pl.when(b == 0)
    def _init():
        det = jnp.maximum(ucovxx[...] * ucovyy[...] - ucovxy[...] * ucovxy[...], 1e-8)
        mid = 0.5 * (ucovxx[...] + ucovyy[...])
        lam1 = mid + jnp.sqrt(jnp.maximum(mid * mid - det, 0.1))
        radii_out[...] = jnp.ceil(3.0 * jnp.sqrt(lam1)).astype(jnp.int32)

    lane = lax.broadcasted_iota(jnp.int32, (1, P), 1)
    px = (lane % W).astype(jnp.float32) + 0.5
    py = (b * BAND_ROWS + lane // W).astype(jnp.float32) + 0.5

    ri = lax.broadcasted_iota(jnp.int32, (K, K), 0)
    ci = lax.broadcasted_iota(jnp.int32, (K, K), 1)
    mstrict = (ri > ci).astype(jnp.float32)
    iota_k1 = lax.broadcasted_iota(jnp.int32, (K, 1), 0)

    cnt = cnt_ref[b]
    trips = lax.div(cnt + (K - 1), K)

    def body(i, carry):
        logT, img = carry
        base = i * K
        gsl = geom[0, pl.ds(base, K), :]
        dx = px - gsl[:, 0:1]
        dy = py - gsl[:, 1:2]
        power = (gsl[:, 2:3] * (dx * dx) + gsl[:, 4:5] * (dy * dy)
                 + gsl[:, 3:4] * (dx * dy))
        power = jnp.minimum(power, 0.0)
        araw = jnp.minimum(0.99, jnp.exp(power + gsl[:, 5:6]))
        m = jnp.logical_and(araw > ALPHA_MIN, base + iota_k1 < cnt)
        alpha = jnp.where(m, araw, 0.0)

        cntv = jnp.sum(m.astype(jnp.float32), axis=1, keepdims=True)
        pcp_out[0, pl.ds(base, K), :] = cntv.astype(jnp.int32)

        logm = jnp.log1p(-alpha)
        s_excl = jnp.dot(mstrict, logm, preferred_element_type=jnp.float32)
        w = jnp.exp(logT + s_excl) * alpha
        ctc = colp[0, i, 0:3, :]
        img = img + jnp.dot(ctc, w, preferred_element_type=jnp.float32)
        logT = logT + jnp.sum(logm, axis=0, keepdims=True)
        return logT, img

    logT = jnp.zeros((1, P), jnp.float32)
    img = jnp.zeros((3, P), jnp.float32)
    logT, img = lax.fori_loop(0, trips, body, (logT, img))

    img = img + bg[...] * jnp.exp(logT)
    for r in range(BAND_ROWS):
        color_out[:, r, :] = img[:, r * W:(r + 1) * W]


@jax.jit
def kernel(means2D, colors, opacities, scales, rotations, depths, background):
    order = jnp.argsort(depths)
    mx = means2D[order, 0]
    my = means2D[order, 1]
    ssx = scales[order, 0]
    ssy = scales[order, 1]
    srot = rotations[order]
    sop = opacities[order, 0]
    scol = colors[order]

    a = jnp.cos(srot)
    bb = jnp.sin(srot)
    sx2 = ssx * ssx
    sy2 = ssy * ssy
    cov_xx = a * a * sx2 + bb * bb * sy2
    cov_xy = a * bb * (sx2 - sy2)
    cov_yy = bb * bb * sx2 + a * a * sy2
    det = jnp.maximum(cov_xx * cov_yy - cov_xy * cov_xy, 1e-8)
    ca = -0.5 * cov_yy / det
    cb = cov_xy / det
    cc = -0.5 * cov_xx / det
    lop = jnp.log(sop)

    mid = 0.5 * (cov_xx + cov_yy)
    lam1 = mid + jnp.sqrt(jnp.maximum(mid * mid - det, 0.1))
    r2 = 2.0 * lam1 * (jnp.log(255.0 * sop) + CULL_SLACK)
    reach = jnp.sqrt(jnp.maximum(r2, 0.0))
    dead = r2 <= 0.0
    ylo = jnp.where(dead, jnp.inf, my - reach)
    yhi = jnp.where(dead, -jnp.inf, my + reach)

    blo = jnp.arange(NBANDS, dtype=jnp.float32)[:, None] * BAND_ROWS + 0.5
    bhi = blo + (BAND_ROWS - 1.0)
    member = (ylo[None, :] <= bhi) & (yhi[None, :] >= blo)
    cnt = member.sum(axis=1).astype(jnp.int32)
    perm = jnp.argsort(~member, axis=1, stable=True)

    geomsrc = jnp.stack(
        [mx, my, ca, cb, cc, lop,
         jnp.zeros_like(mx), jnp.zeros_like(mx)], axis=1)
    geom = geomsrc[perm]
    colpk = scol[perm]
    colpk = jnp.pad(colpk, ((0, 0), (0, 0), (0, 1)))
    colp = colpk.reshape(NBANDS, NCK, K, 4).transpose(0, 1, 3, 2)

    ua = jnp.cos(rotations)
    ub = jnp.sin(rotations)
    usx2 = scales[:, 0] ** 2
    usy2 = scales[:, 1] ** 2
    ucovxx = (ua * ua * usx2 + ub * ub * usy2).reshape(N, 1)
    ucovxy = (ua * ub * (usx2 - usy2)).reshape(N, 1)
    ucovyy = (ub * ub * usx2 + ua * ua * usy2).reshape(N, 1)

    full = lambda shape: pl.BlockSpec(shape, lambda i, s: (0,) * len(shape))
    grid_spec = pltpu.PrefetchScalarGridSpec(
        num_scalar_prefetch=1,
        grid=(NBANDS,),
        in_specs=[
            pl.BlockSpec((1, N, 8), lambda i, s: (i, 0, 0)),
            pl.BlockSpec((1, NCK, 4, K), lambda i, s: (i, 0, 0, 0)),
            full((N, 1)), full((N, 1)), full((N, 1)), full((3, 1)),
        ],
        out_specs=[
            pl.BlockSpec((3, BAND_ROWS, W), lambda i, s: (0, i, 0)),
            full((N, 1)),
            pl.BlockSpec((1, N, 1), lambda i, s: (i, 0, 0)),
        ],
    )
    color, radii, pcp = pl.pallas_call(
        _raster_kernel,
        grid_spec=grid_spec,
        out_shape=[
            jax.ShapeDtypeStruct((3, H, W), jnp.float32),
            jax.ShapeDtypeStruct((N, 1), jnp.int32),
            jax.ShapeDtypeStruct((NBANDS, N, 1), jnp.int32),
        ],
    )(cnt, geom, colp, ucovxx, ucovxy, ucovyy, background.reshape(3, 1))

    valid = jnp.arange(N, dtype=jnp.int32)[None, :] < cnt[:, None]
    tgt = jnp.where(valid, perm, N)
    pc_sorted = jnp.zeros((N,), jnp.int32).at[tgt.reshape(-1)].add(
        pcp.reshape(-1), mode="drop")
    pix_covered = jnp.zeros((N,), jnp.int32).at[order].set(pc_sorted)
    return color, radii.reshape(N), pix_covered


# speedup vs baseline: 1.8750x; 1.8750x over previous
import functools

import jax
import jax.numpy as jnp
from jax import lax
from jax.experimental import pallas as pl
from jax.experimental.pallas import tpu as pltpu
from jax.experimental.pallas import tpu_sc as plsc

H = 128
W = 128
N = 2048

BAND_ROWS = 8
NBANDS = H // BAND_ROWS
P = BAND_ROWS * W
K = 128
NCK = N // K
CHROWS = 16
ALPHA_MIN = 1.0 / 255.0
CULL_SLACK = 0.01
SC_L = 16


def _sc_bin_kernel(chans, czero, geom_o, colp_o, cnt_o,
                   chans_v, geo_v, col_v, cnt_v):
    band = lax.axis_index("s") * 2 + lax.axis_index("c")

    @pl.when(band < NBANDS)
    def _():
        pltpu.sync_copy(chans, chans_v)
        pltpu.sync_copy(czero, col_v)
        blo_s = band.astype(jnp.float32) * BAND_ROWS + 0.5
        bhi_s = blo_s + (BAND_ROWS - 1.0)

        def chunk(i, offv):
            ones = jnp.full((SC_L,), 1, jnp.int32)
            zeros = jnp.zeros((SC_L,), jnp.int32)
            idx = (jnp.broadcast_to(i * SC_L, (SC_L,))
                   + lax.broadcasted_iota(jnp.int32, (SC_L,), 0))
            ylo = plsc.load_gather(chans_v, [idx])
            yhi = plsc.load_gather(chans_v, [idx + jnp.full((SC_L,), N, jnp.int32)])
            m = jnp.logical_and(ylo <= jnp.broadcast_to(bhi_s, (SC_L,)),
                                yhi >= jnp.broadcast_to(blo_s, (SC_L,)))
            mi = jnp.where(m, ones, zeros)
            pos = offv + plsc.cumsum(mi) - ones
            pos8 = pos * jnp.full((SC_L,), 8, jnp.int32)
            for ch in range(8):
                v = plsc.load_gather(
                    chans_v, [idx + jnp.full((SC_L,), (2 + ch) * N, jnp.int32)])
                plsc.store_scatter(
                    geo_v, [pos8 + jnp.full((SC_L,), ch, jnp.int32)], v, mask=m)
            bank = lax.shift_right_logical(pos, jnp.full((SC_L,), 7, jnp.int32))
            lanep = jnp.bitwise_and(pos, jnp.full((SC_L,), 127, jnp.int32))
            cbase = bank * jnp.full((SC_L,), 4 * K, jnp.int32) + lanep
            for j in range(3):
                v = plsc.load_gather(
                    chans_v, [idx + jnp.full((SC_L,), (10 + j) * N, jnp.int32)])
                plsc.store_scatter(
                    col_v, [cbase + jnp.full((SC_L,), j * K, jnp.int32)],
                    v, mask=m)
            return offv + plsc.all_reduce_population_count(m)

        offv = lax.fori_loop(0, N // SC_L, chunk,
                             jnp.zeros((SC_L,), jnp.int32))
        lane = lax.broadcasted_iota(jnp.int32, (SC_L,), 0)
        cnt_v[...] = jnp.where(lane == jnp.zeros((SC_L,), jnp.int32),
                               offv, jnp.zeros((SC_L,), jnp.int32))
        pltpu.sync_copy(geo_v, geom_o.at[band])
        pltpu.sync_copy(col_v, colp_o.at[band])
        pltpu.sync_copy(cnt_v, cnt_o.at[band])


_sc_bin = pl.kernel(
    _sc_bin_kernel,
    mesh=plsc.VectorSubcoreMesh(core_axis_name="c", subcore_axis_name="s"),
    compiler_params=pltpu.CompilerParams(needs_layout_passes=False),
    out_type=[
        jax.ShapeDtypeStruct((NBANDS, N * 8), jnp.float32),
        jax.ShapeDtypeStruct((NBANDS, NCK * 4 * K), jnp.float32),
        jax.ShapeDtypeStruct((NBANDS, SC_L), jnp.int32),
    ],
    scratch_types=[
        pltpu.VMEM((CHROWS * N,), jnp.float32),
        pltpu.VMEM((N * 8,), jnp.float32),
        pltpu.VMEM((NCK * 4 * K,), jnp.float32),
        pltpu.VMEM((SC_L,), jnp.int32),
    ],
)


def _raster_kernel(cnt_ref, geom, colp, ucovxx, ucovxy, ucovyy, bg,
                   color_out, radii_out, pcp_out):
    b = pl.program_id(0)

    @pl.when(b == 0)
    def _init():
        det = jnp.maximum(ucovxx[...] * ucovyy[...] - ucovxy[...] * ucovxy[...], 1e-8)
        mid = 0.5 * (ucovxx[...] + ucovyy[...])
        lam1 = mid + jnp.sqrt(jnp.maximum(mid * mid - det, 0.1))
        radii_out[...] = jnp.ceil(3.0 * jnp.sqrt(lam1)).astype(jnp.int32)

    lane = lax.broadcasted_iota(jnp.int32, (1, P), 1)
    px = (lane % W).astype(jnp.float32) + 0.5
    py = (b * BAND_ROWS + lane // W).astype(jnp.float32) + 0.5

    ri = lax.broadcasted_iota(jnp.int32, (K, K), 0)
    ci = lax.broadcasted_iota(jnp.int32, (K, K), 1)
    mstrict = (ri > ci).astype(jnp.float32)
    iota_k1 = lax.broadcasted_iota(jnp.int32, (K, 1), 0)

    cnt = cnt_ref[b]
    trips = lax.div(cnt + (K - 1), K)

    def body(i, carry):
        logT, img = carry
        base = i * K
        gsl = geom[0, pl.ds(base, K), :]
        dx = px - gsl[:, 0:1]
        dy = py - gsl[:, 1:2]
        power = (gsl[:, 2:3] * (dx * dx) + gsl[:, 4:5] * (dy * dy)
                 + gsl[:, 3:4] * (dx * dy))
        power = jnp.minimum(power, 0.0)
        araw = jnp.minimum(0.99, jnp.exp(power + gsl[:, 5:6]))
        m = jnp.logical_and(araw > ALPHA_MIN, base + iota_k1 < cnt)
        alpha = jnp.where(m, araw, 0.0)

        cntv = jnp.sum(m.astype(jnp.float32), axis=1, keepdims=True)
        pcp_out[0, pl.ds(base, K), :] = cntv.astype(jnp.int32)

        logm = jnp.log1p(-alpha)
        s_excl = jnp.dot(mstrict, logm, preferred_element_type=jnp.float32)
        w = jnp.exp(logT + s_excl) * alpha
        ctc = colp[0, i, 0:3, :]
        img = img + jnp.dot(ctc, w, preferred_element_type=jnp.float32)
        logT = logT + jnp.sum(logm, axis=0, keepdims=True)
        return logT, img

    logT = jnp.zeros((1, P), jnp.float32)
    img = jnp.zeros((3, P), jnp.float32)
    logT, img = lax.fori_loop(0, trips, body, (logT, img))

    img = img + bg[...] * jnp.exp(logT)
    for r in range(BAND_ROWS):
        color_out[:, r, :] = img[:, r * W:(r + 1) * W]


@jax.jit
def kernel(means2D, colors, opacities, scales, rotations, depths, background):
    order = jnp.argsort(depths)
    mx = means2D[order, 0]
    my = means2D[order, 1]
    ssx = scales[order, 0]
    ssy = scales[order, 1]
    srot = rotations[order]
    sop = opacities[order, 0]
    scol = colors[order]

    a = jnp.cos(srot)
    bb = jnp.sin(srot)
    sx2 = ssx * ssx
    sy2 = ssy * ssy
    cov_xx = a * a * sx2 + bb * bb * sy2
    cov_xy = a * bb * (sx2 - sy2)
    cov_yy = bb * bb * sx2 + a * a * sy2
    det = jnp.maximum(cov_xx * cov_yy - cov_xy * cov_xy, 1e-8)
    ca = -0.5 * cov_yy / det
    cb = cov_xy / det
    cc = -0.5 * cov_xx / det
    lop = jnp.log(sop)

    mid = 0.5 * (cov_xx + cov_yy)
    lam1 = mid + jnp.sqrt(jnp.maximum(mid * mid - det, 0.1))
    r2 = 2.0 * lam1 * (jnp.log(255.0 * sop) + CULL_SLACK)
    reach = jnp.sqrt(jnp.maximum(r2, 0.0))
    dead = r2 <= 0.0
    ylo = jnp.where(dead, jnp.inf, my - reach)
    yhi = jnp.where(dead, -jnp.inf, my + reach)

    gidf = jnp.arange(N, dtype=jnp.float32)
    zr = jnp.zeros_like(mx)
    chans = jnp.stack(
        [ylo, yhi, mx, my, ca, cb, cc, lop, gidf, zr,
         scol[:, 0], scol[:, 1], scol[:, 2], zr, zr, zr], axis=0)
    czero = jnp.zeros((NCK * 4 * K,), jnp.float32)

    geomf, colpf, cnt16 = _sc_bin(chans.reshape(-1), czero)
    geom = geomf.reshape(NBANDS, N, 8)
    colp = colpf.reshape(NBANDS, NCK, 4, K)
    cnt = cnt16[:, 0]

    ua = jnp.cos(rotations)
    ub = jnp.sin(rotations)
    usx2 = scales[:, 0] ** 2
    usy2 = scales[:, 1] ** 2
    ucovxx = (ua * ua * usx2 + ub * ub * usy2).reshape(N, 1)
    ucovxy = (ua * ub * (usx2 - usy2)).reshape(N, 1)
    ucovyy = (ub * ub * usx2 + ua * ua * usy2).reshape(N, 1)

    full = lambda shape: pl.BlockSpec(shape, lambda i, s: (0,) * len(shape))
    grid_spec = pltpu.PrefetchScalarGridSpec(
        num_scalar_prefetch=1,
        grid=(NBANDS,),
        in_specs=[
            pl.BlockSpec((1, N, 8), lambda i, s: (i, 0, 0)),
            pl.BlockSpec((1, NCK, 4, K), lambda i, s: (i, 0, 0, 0)),
            full((N, 1)), full((N, 1)), full((N, 1)), full((3, 1)),
        ],
        out_specs=[
            pl.BlockSpec((3, BAND_ROWS, W), lambda i, s: (0, i, 0)),
            full((N, 1)),
            pl.BlockSpec((1, N, 1), lambda i, s: (i, 0, 0)),
        ],
    )
    color, radii, pcp = pl.pallas_call(
        _raster_kernel,
        grid_spec=grid_spec,
        out_shape=[
            jax.ShapeDtypeStruct((3, H, W), jnp.float32),
            jax.ShapeDtypeStruct((N, 1), jnp.int32),
            jax.ShapeDtypeStruct((NBANDS, N, 1), jnp.int32),
        ],
    )(cnt, geom, colp, ucovxx, ucovxy, ucovyy, background.reshape(3, 1))

    gid = geom[:, :, 6].astype(jnp.int32)
    valid = jnp.arange(N, dtype=jnp.int32)[None, :] < cnt[:, None]
    tgt = jnp.where(valid, gid, N)
    pc_sorted = jnp.zeros((N,), jnp.int32).at[tgt.reshape(-1)].add(
        pcp.reshape(-1), mode="drop")
    pix_covered = jnp.zeros((N,), jnp.int32).at[order].set(pc_sorted)
    return color, radii.reshape(N), pix_covered


# speedup vs baseline: 2.2793x; 1.2156x over previous
import functools

import jax
import jax.numpy as jnp
from jax import lax
from jax.experimental import pallas as pl
from jax.experimental.pallas import tpu as pltpu
from jax.experimental.pallas import tpu_sc as plsc

H = 128
W = 128
N = 2048

BAND_ROWS = 8
NBANDS = H // BAND_ROWS
P = BAND_ROWS * W
K = 128
NCK = N // K
CHROWS = 16
ALPHA_MIN = 1.0 / 255.0
CULL_SLACK = 0.01
SC_L = 16


def _sc_bin_kernel(chans, czero, geom_o, colp_o, cnt_o,
                   chans_v, geo_v, col_v, cnt_v):
    band = lax.axis_index("s") * 2 + lax.axis_index("c")

    @pl.when(band < NBANDS)
    def _():
        pltpu.sync_copy(chans, chans_v)
        pltpu.sync_copy(czero, col_v)
        blo_s = band.astype(jnp.float32) * BAND_ROWS + 0.5
        bhi_s = blo_s + (BAND_ROWS - 1.0)

        def chunk(i, offv):
            ones = jnp.full((SC_L,), 1, jnp.int32)
            zeros = jnp.zeros((SC_L,), jnp.int32)
            idx = (jnp.broadcast_to(i * SC_L, (SC_L,))
                   + lax.broadcasted_iota(jnp.int32, (SC_L,), 0))
            ylo = plsc.load_gather(chans_v, [idx])
            yhi = plsc.load_gather(chans_v, [idx + jnp.full((SC_L,), N, jnp.int32)])
            m = jnp.logical_and(ylo <= jnp.broadcast_to(bhi_s, (SC_L,)),
                                yhi >= jnp.broadcast_to(blo_s, (SC_L,)))
            mi = jnp.where(m, ones, zeros)
            pos = offv + plsc.cumsum(mi) - ones
            pos8 = pos * jnp.full((SC_L,), 8, jnp.int32)
            for ch in range(8):
                v = plsc.load_gather(
                    chans_v, [idx + jnp.full((SC_L,), (2 + ch) * N, jnp.int32)])
                plsc.store_scatter(
                    geo_v, [pos8 + jnp.full((SC_L,), ch, jnp.int32)], v, mask=m)
            bank = lax.shift_right_logical(pos, jnp.full((SC_L,), 7, jnp.int32))
            lanep = jnp.bitwise_and(pos, jnp.full((SC_L,), 127, jnp.int32))
            cbase = bank * jnp.full((SC_L,), 4 * K, jnp.int32) + lanep
            for j in range(3):
                v = plsc.load_gather(
                    chans_v, [idx + jnp.full((SC_L,), (10 + j) * N, jnp.int32)])
                plsc.store_scatter(
                    col_v, [cbase + jnp.full((SC_L,), j * K, jnp.int32)],
                    v, mask=m)
            return offv + plsc.all_reduce_population_count(m)

        offv = lax.fori_loop(0, N // SC_L, chunk,
                             jnp.zeros((SC_L,), jnp.int32))
        lane = lax.broadcasted_iota(jnp.int32, (SC_L,), 0)
        cnt_v[...] = jnp.where(lane == jnp.zeros((SC_L,), jnp.int32),
                               offv, jnp.zeros((SC_L,), jnp.int32))
        pltpu.sync_copy(geo_v, geom_o.at[band])
        pltpu.sync_copy(col_v, colp_o.at[band])
        pltpu.sync_copy(cnt_v, cnt_o.at[band])


_sc_bin = pl.kernel(
    _sc_bin_kernel,
    mesh=plsc.VectorSubcoreMesh(core_axis_name="c", subcore_axis_name="s"),
    compiler_params=pltpu.CompilerParams(needs_layout_passes=False),
    out_type=[
        jax.ShapeDtypeStruct((NBANDS, N * 8), jnp.float32),
        jax.ShapeDtypeStruct((NBANDS, NCK * 4 * K), jnp.float32),
        jax.ShapeDtypeStruct((NBANDS, SC_L), jnp.int32),
    ],
    scratch_types=[
        pltpu.VMEM((CHROWS * N,), jnp.float32),
        pltpu.VMEM((N * 8,), jnp.float32),
        pltpu.VMEM((NCK * 4 * K,), jnp.float32),
        pltpu.VMEM((SC_L,), jnp.int32),
    ],
)


def _raster_kernel(cnt_ref, geom, colp, ucovxx, ucovxy, ucovyy, bg,
                   color_out, radii_out, pcp_out):
    b = pl.program_id(0)

    @pl.when(b == 0)
    def _init():
        det = jnp.maximum(ucovxx[...] * ucovyy[...] - ucovxy[...] * ucovxy[...], 1e-8)
        mid = 0.5 * (ucovxx[...] + ucovyy[...])
        lam1 = mid + jnp.sqrt(jnp.maximum(mid * mid - det, 0.1))
        radii_out[...] = jnp.ceil(3.0 * jnp.sqrt(lam1)).astype(jnp.int32)

    lane = lax.broadcasted_iota(jnp.int32, (1, P), 1)
    px = (lane % W).astype(jnp.float32) + 0.5
    py = (b * BAND_ROWS + lane // W).astype(jnp.float32) + 0.5

    ri = lax.broadcasted_iota(jnp.int32, (K, K), 0)
    ci = lax.broadcasted_iota(jnp.int32, (K, K), 1)
    mstrict = (ri > ci).astype(jnp.float32)
    iota_k1 = lax.broadcasted_iota(jnp.int32, (K, 1), 0)

    cnt = cnt_ref[b]
    trips = lax.div(cnt + (K - 1), K)

    def body(i, carry):
        logT, img = carry
        base = i * K
        gsl = geom[0, pl.ds(base, K), :]
        dx = px - gsl[:, 0:1]
        dy = py - gsl[:, 1:2]
        power = (gsl[:, 2:3] * (dx * dx) + gsl[:, 4:5] * (dy * dy)
                 + gsl[:, 3:4] * (dx * dy))
        power = jnp.minimum(power, 0.0)
        araw = jnp.minimum(0.99, jnp.exp(power + gsl[:, 5:6]))
        m = jnp.logical_and(araw > ALPHA_MIN, base + iota_k1 < cnt)
        alpha = jnp.where(m, araw, 0.0)

        cntv = jnp.sum(m.astype(jnp.float32), axis=1, keepdims=True)
        pcp_out[0, pl.ds(base, K), :] = cntv.astype(jnp.int32)

        logm = jnp.log1p(-alpha)
        s_excl = jnp.dot(mstrict, logm, preferred_element_type=jnp.float32)
        w = jnp.exp(logT + s_excl) * alpha
        ctc = colp[0, i, 0:3, :]
        img = img + jnp.dot(ctc, w, preferred_element_type=jnp.float32)
        logT = logT + jnp.sum(logm, axis=0, keepdims=True)
        return logT, img

    logT = jnp.zeros((1, P), jnp.float32)
    img = jnp.zeros((3, P), jnp.float32)
    logT, img = lax.fori_loop(0, trips, body, (logT, img))

    img = img + bg[...] * jnp.exp(logT)
    for r in range(BAND_ROWS):
        color_out[:, r, :] = img[:, r * W:(r + 1) * W]


@jax.jit
def kernel(means2D, colors, opacities, scales, rotations, depths, background):
    order = jnp.argsort(depths)

    mx = means2D[:, 0]
    my = means2D[:, 1]
    op0 = opacities[:, 0]
    a = jnp.cos(rotations)
    bb = jnp.sin(rotations)
    sx2 = scales[:, 0] ** 2
    sy2 = scales[:, 1] ** 2
    cov_xx = a * a * sx2 + bb * bb * sy2
    cov_xy = a * bb * (sx2 - sy2)
    cov_yy = bb * bb * sx2 + a * a * sy2
    det = jnp.maximum(cov_xx * cov_yy - cov_xy * cov_xy, 1e-8)
    ca = -0.5 * cov_yy / det
    cb = cov_xy / det
    cc = -0.5 * cov_xx / det
    lop = jnp.log(op0)

    mid = 0.5 * (cov_xx + cov_yy)
    lam1 = mid + jnp.sqrt(jnp.maximum(mid * mid - det, 0.1))
    r2 = 2.0 * lam1 * (jnp.log(255.0 * op0) + CULL_SLACK)
    reach = jnp.sqrt(jnp.maximum(r2, 0.0))
    dead = r2 <= 0.0
    ylo = jnp.where(dead, jnp.inf, my - reach)
    yhi = jnp.where(dead, -jnp.inf, my + reach)

    gidf = jnp.arange(N, dtype=jnp.float32)
    zr = jnp.zeros_like(mx)
    chans_u = jnp.stack(
        [ylo, yhi, mx, my, ca, cb, cc, lop, gidf, zr,
         colors[:, 0], colors[:, 1], colors[:, 2], zr, zr, zr], axis=0)
    chans = chans_u[:, order]
    czero = jnp.zeros((NCK * 4 * K,), jnp.float32)

    geomf, colpf, cnt16 = _sc_bin(chans.reshape(-1), czero)
    geom = geomf.reshape(NBANDS, N, 8)
    colp = colpf.reshape(NBANDS, NCK, 4, K)
    cnt = cnt16[:, 0]

    ucovxx = cov_xx.reshape(N, 1)
    ucovxy = cov_xy.reshape(N, 1)
    ucovyy = cov_yy.reshape(N, 1)

    full = lambda shape: pl.BlockSpec(shape, lambda i, s: (0,) * len(shape))
    grid_spec = pltpu.PrefetchScalarGridSpec(
        num_scalar_prefetch=1,
        grid=(NBANDS,),
        in_specs=[
            pl.BlockSpec((1, N, 8), lambda i, s: (i, 0, 0)),
            pl.BlockSpec((1, NCK, 4, K), lambda i, s: (i, 0, 0, 0)),
            full((N, 1)), full((N, 1)), full((N, 1)), full((3, 1)),
        ],
        out_specs=[
            pl.BlockSpec((3, BAND_ROWS, W), lambda i, s: (0, i, 0)),
            full((N, 1)),
            pl.BlockSpec((1, N, 1), lambda i, s: (i, 0, 0)),
        ],
    )
    color, radii, pcp = pl.pallas_call(
        _raster_kernel,
        grid_spec=grid_spec,
        out_shape=[
            jax.ShapeDtypeStruct((3, H, W), jnp.float32),
            jax.ShapeDtypeStruct((N, 1), jnp.int32),
            jax.ShapeDtypeStruct((NBANDS, N, 1), jnp.int32),
        ],
    )(cnt, geom, colp, ucovxx, ucovxy, ucovyy, background.reshape(3, 1))

    gid = geom[:, :, 6].astype(jnp.int32)
    valid = jnp.arange(N, dtype=jnp.int32)[None, :] < cnt[:, None]
    tgt = jnp.where(valid, gid, N)
    pix_covered = jnp.zeros((N,), jnp.int32).at[tgt.reshape(-1)].add(
        pcp.reshape(-1), mode="drop")
    return color, radii.reshape(N), pix_covered


# speedup vs baseline: 2.3448x; 1.0287x over previous
import functools

import jax
import jax.numpy as jnp
from jax import lax
from jax.experimental import pallas as pl
from jax.experimental.pallas import tpu as pltpu
from jax.experimental.pallas import tpu_sc as plsc

H = 128
W = 128
N = 2048

BAND_ROWS = 8
NBANDS = H // BAND_ROWS
P = BAND_ROWS * W
K = 256
NCK = N // K
CHROWS = 16
ALPHA_MIN = 1.0 / 255.0
CULL_SLACK = 0.01
SC_L = 16


def _sc_bin_kernel(chans, czero, geom_o, colp_o, cnt_o,
                   chans_v, geo_v, col_v, cnt_v):
    band = lax.axis_index("s") * 2 + lax.axis_index("c")

    @pl.when(band < NBANDS)
    def _():
        pltpu.sync_copy(chans, chans_v)
        pltpu.sync_copy(czero, col_v)
        blo_s = band.astype(jnp.float32) * BAND_ROWS + 0.5
        bhi_s = blo_s + (BAND_ROWS - 1.0)

        def chunk(i, offv):
            ones = jnp.full((SC_L,), 1, jnp.int32)
            zeros = jnp.zeros((SC_L,), jnp.int32)
            idx = (jnp.broadcast_to(i * SC_L, (SC_L,))
                   + lax.broadcasted_iota(jnp.int32, (SC_L,), 0))
            ylo = plsc.load_gather(chans_v, [idx])
            yhi = plsc.load_gather(chans_v, [idx + jnp.full((SC_L,), N, jnp.int32)])
            m = jnp.logical_and(ylo <= jnp.broadcast_to(bhi_s, (SC_L,)),
                                yhi >= jnp.broadcast_to(blo_s, (SC_L,)))
            mi = jnp.where(m, ones, zeros)
            pos = offv + plsc.cumsum(mi) - ones
            pos8 = pos * jnp.full((SC_L,), 8, jnp.int32)
            for ch in range(8):
                v = plsc.load_gather(
                    chans_v, [idx + jnp.full((SC_L,), (2 + ch) * N, jnp.int32)])
                plsc.store_scatter(
                    geo_v, [pos8 + jnp.full((SC_L,), ch, jnp.int32)], v, mask=m)
            bank = lax.shift_right_logical(pos, jnp.full((SC_L,), 8, jnp.int32))
            lanep = jnp.bitwise_and(pos, jnp.full((SC_L,), 255, jnp.int32))
            cbase = bank * jnp.full((SC_L,), 4 * K, jnp.int32) + lanep
            for j in range(3):
                v = plsc.load_gather(
                    chans_v, [idx + jnp.full((SC_L,), (10 + j) * N, jnp.int32)])
                plsc.store_scatter(
                    col_v, [cbase + jnp.full((SC_L,), j * K, jnp.int32)],
                    v, mask=m)
            return offv + plsc.all_reduce_population_count(m)

        offv = lax.fori_loop(0, N // SC_L, chunk,
                             jnp.zeros((SC_L,), jnp.int32))
        lane = lax.broadcasted_iota(jnp.int32, (SC_L,), 0)
        cnt_v[...] = jnp.where(lane == jnp.zeros((SC_L,), jnp.int32),
                               offv, jnp.zeros((SC_L,), jnp.int32))
        pltpu.sync_copy(geo_v, geom_o.at[band])
        pltpu.sync_copy(col_v, colp_o.at[band])
        pltpu.sync_copy(cnt_v, cnt_o.at[band])


_sc_bin = pl.kernel(
    _sc_bin_kernel,
    mesh=plsc.VectorSubcoreMesh(core_axis_name="c", subcore_axis_name="s"),
    compiler_params=pltpu.CompilerParams(needs_layout_passes=False),
    out_type=[
        jax.ShapeDtypeStruct((NBANDS, N * 8), jnp.float32),
        jax.ShapeDtypeStruct((NBANDS, NCK * 4 * K), jnp.float32),
        jax.ShapeDtypeStruct((NBANDS, SC_L), jnp.int32),
    ],
    scratch_types=[
        pltpu.VMEM((CHROWS * N,), jnp.float32),
        pltpu.VMEM((N * 8,), jnp.float32),
        pltpu.VMEM((NCK * 4 * K,), jnp.float32),
        pltpu.VMEM((SC_L,), jnp.int32),
    ],
)


def _raster_kernel(cnt_ref, geom, colp, ucovxx, ucovxy, ucovyy, bg,
                   color_out, radii_out, pcp_out):
    b = pl.program_id(0)

    @pl.when(b == 0)
    def _init():
        det = jnp.maximum(ucovxx[...] * ucovyy[...] - ucovxy[...] * ucovxy[...], 1e-8)
        mid = 0.5 * (ucovxx[...] + ucovyy[...])
        lam1 = mid + jnp.sqrt(jnp.maximum(mid * mid - det, 0.1))
        radii_out[...] = jnp.ceil(3.0 * jnp.sqrt(lam1)).astype(jnp.int32)

    lane = lax.broadcasted_iota(jnp.int32, (1, P), 1)
    px = (lane % W).astype(jnp.float32) + 0.5
    py = (b * BAND_ROWS + lane // W).astype(jnp.float32) + 0.5

    ri = lax.broadcasted_iota(jnp.int32, (K, K), 0)
    ci = lax.broadcasted_iota(jnp.int32, (K, K), 1)
    mstrict = (ri > ci).astype(jnp.float32)
    iota_k1 = lax.broadcasted_iota(jnp.int32, (K, 1), 0)

    cnt = cnt_ref[b]
    trips = lax.div(cnt + (K - 1), K)

    def body(i, carry):
        logT, img = carry
        base = i * K
        gsl = geom[0, pl.ds(base, K), :]
        dx = px - gsl[:, 0:1]
        dy = py - gsl[:, 1:2]
        power = (gsl[:, 2:3] * (dx * dx) + gsl[:, 4:5] * (dy * dy)
                 + gsl[:, 3:4] * (dx * dy))
        power = jnp.minimum(power, 0.0)
        araw = jnp.minimum(0.99, jnp.exp(power + gsl[:, 5:6]))
        m = jnp.logical_and(araw > ALPHA_MIN, base + iota_k1 < cnt)
        alpha = jnp.where(m, araw, 0.0)

        cntv = jnp.sum(m.astype(jnp.float32), axis=1, keepdims=True)
        pcp_out[0, pl.ds(base, K), :] = cntv.astype(jnp.int32)

        logm = jnp.log1p(-alpha)
        s_excl = jnp.dot(mstrict, logm, preferred_element_type=jnp.float32)
        w = jnp.exp(logT + s_excl) * alpha
        ctc = colp[0, i, 0:3, :]
        img = img + jnp.dot(ctc, w, preferred_element_type=jnp.float32)
        logT = logT + jnp.sum(logm, axis=0, keepdims=True)
        return logT, img

    logT = jnp.zeros((1, P), jnp.float32)
    img = jnp.zeros((3, P), jnp.float32)
    logT, img = lax.fori_loop(0, trips, body, (logT, img))

    img = img + bg[...] * jnp.exp(logT)
    for r in range(BAND_ROWS):
        color_out[:, r, :] = img[:, r * W:(r + 1) * W]


@jax.jit
def kernel(means2D, colors, opacities, scales, rotations, depths, background):
    order = jnp.argsort(depths)

    mx = means2D[:, 0]
    my = means2D[:, 1]
    op0 = opacities[:, 0]
    a = jnp.cos(rotations)
    bb = jnp.sin(rotations)
    sx2 = scales[:, 0] ** 2
    sy2 = scales[:, 1] ** 2
    cov_xx = a * a * sx2 + bb * bb * sy2
    cov_xy = a * bb * (sx2 - sy2)
    cov_yy = bb * bb * sx2 + a * a * sy2
    det = jnp.maximum(cov_xx * cov_yy - cov_xy * cov_xy, 1e-8)
    ca = -0.5 * cov_yy / det
    cb = cov_xy / det
    cc = -0.5 * cov_xx / det
    lop = jnp.log(op0)

    mid = 0.5 * (cov_xx + cov_yy)
    lam1 = mid + jnp.sqrt(jnp.maximum(mid * mid - det, 0.1))
    r2 = 2.0 * lam1 * (jnp.log(255.0 * op0) + CULL_SLACK)
    reach = jnp.sqrt(jnp.maximum(r2, 0.0))
    dead = r2 <= 0.0
    ylo = jnp.where(dead, jnp.inf, my - reach)
    yhi = jnp.where(dead, -jnp.inf, my + reach)

    gidf = jnp.arange(N, dtype=jnp.float32)
    zr = jnp.zeros_like(mx)
    chans_u = jnp.stack(
        [ylo, yhi, mx, my, ca, cb, cc, lop, gidf, zr,
         colors[:, 0], colors[:, 1], colors[:, 2], zr, zr, zr], axis=0)
    chans = chans_u[:, order]
    czero = jnp.zeros((NCK * 4 * K,), jnp.float32)

    geomf, colpf, cnt16 = _sc_bin(chans.reshape(-1), czero)
    geom = geomf.reshape(NBANDS, N, 8)
    colp = colpf.reshape(NBANDS, NCK, 4, K)
    cnt = cnt16[:, 0]

    ucovxx = cov_xx.reshape(N, 1)
    ucovxy = cov_xy.reshape(N, 1)
    ucovyy = cov_yy.reshape(N, 1)

    full = lambda shape: pl.BlockSpec(shape, lambda i, s: (0,) * len(shape))
    grid_spec = pltpu.PrefetchScalarGridSpec(
        num_scalar_prefetch=1,
        grid=(NBANDS,),
        in_specs=[
            pl.BlockSpec((1, N, 8), lambda i, s: (i, 0, 0)),
            pl.BlockSpec((1, NCK, 4, K), lambda i, s: (i, 0, 0, 0)),
            full((N, 1)), full((N, 1)), full((N, 1)), full((3, 1)),
        ],
        out_specs=[
            pl.BlockSpec((3, BAND_ROWS, W), lambda i, s: (0, i, 0)),
            full((N, 1)),
            pl.BlockSpec((1, N, 1), lambda i, s: (i, 0, 0)),
        ],
    )
    color, radii, pcp = pl.pallas_call(
        _raster_kernel,
        grid_spec=grid_spec,
        out_shape=[
            jax.ShapeDtypeStruct((3, H, W), jnp.float32),
            jax.ShapeDtypeStruct((N, 1), jnp.int32),
            jax.ShapeDtypeStruct((NBANDS, N, 1), jnp.int32),
        ],
    )(cnt, geom, colp, ucovxx, ucovxy, ucovyy, background.reshape(3, 1))

    gid = geom[:, :, 6].astype(jnp.int32)
    valid = jnp.arange(N, dtype=jnp.int32)[None, :] < cnt[:, None]
    tgt = jnp.where(valid, gid, N)
    pix_covered = jnp.zeros((N,), jnp.int32).at[tgt.reshape(-1)].add(
        pcp.reshape(-1), mode="drop")
    return color, radii.reshape(N), pix_covered
